# baseline (speedup 1.0000x reference)
import jax
import jax.numpy as jnp
from jax import lax
from jax.experimental import pallas as pl

B, N, T, P, D, O = 16, 2048, 1, 6, 128, 128
BN = B * N
BLK = 512
NPB = N // BLK


def _tc_body(embs_ref, aux_ref, w_ref, b_ref, out_ref):
    e = embs_ref[0, :, 0]
    aux = aux_ref[...].T
    coef = jnp.concatenate([aux[:, 7:8], aux[:, 0:5]], axis=1)
    combined = jnp.sum(e * coef[:, :, None], axis=1)
    y = lax.dot_general(combined, w_ref[...], (((1,), (1,)), ((), ())),
                        preferred_element_type=jnp.float32)
    y = y + aux[:, 5][:, None] * b_ref[...]
    out_ref[0] = jnp.where(aux[:, 6][:, None] > 0, y, 0.0)


@jax.jit
def kernel(embs, vis, W, b, masks):
    visr = vis.reshape(BN, P)
    c = visr[:, 1:].T
    s = 1.0 + jnp.sum(visr[:, 1:], axis=1)[None, :]
    m = masks.reshape(1, BN).astype(jnp.float32)
    aux = jnp.concatenate([c, s, m, jnp.ones((1, BN), jnp.float32)], axis=0)
    b2 = b.reshape(1, O)
    grid = (B, NPB)
    out = pl.pallas_call(
        _tc_body,
        grid=grid,
        in_specs=[
            pl.BlockSpec((1, BLK, 1, P, D), lambda i, j: (i, j, 0, 0, 0)),
            pl.BlockSpec((8, BLK), lambda i, j: (0, i * NPB + j)),
            pl.BlockSpec((O, D), lambda i, j: (0, 0)),
            pl.BlockSpec((1, O), lambda i, j: (0, 0)),
        ],
        out_specs=pl.BlockSpec((1, BLK, O), lambda i, j: (i, j, 0)),
        out_shape=jax.ShapeDtypeStruct((B, N, O), jnp.float32),
    )(embs, aux, W, b2)
    return out
